# Initial kernel scaffold; baseline (speedup 1.0000x reference)
import jax
import jax.numpy as jnp
from jax.experimental import pallas as pl


def kernel(x, edge_index, params):
    raise NotImplementedError("write your pallas kernel here")


# speedup vs baseline: 85.9314x; 85.9314x over previous
import functools

import jax
import jax.numpy as jnp
from jax import lax
from jax.experimental import pallas as pl
from jax.experimental.pallas import tpu as pltpu
from jax.experimental.pallas import tpu_sc as plsc

N = 10000
E = 320000
D_IN = 128
H = 8
C = 8
HID = 64
NC = 2
NS = 16
L = 16
NW = NC * NS
EPW = E // NW
CH = 80
NCHUNK = EPW // CH
WID = 128
NPAD = 10240
WB = NPAD // NS
SROWS = 64


@functools.cache
def _make_sc_edge():
  mesh = plsc.VectorSubcoreMesh(
      core_axis_name="c", subcore_axis_name="s", num_cores=NC, num_subcores=NS)

  @functools.partial(
      pl.kernel,
      out_type=jax.ShapeDtypeStruct((NC, NPAD, WID), jnp.float32),
      mesh=mesh,
      scratch_types=[
          pltpu.VMEM((CH,), jnp.int32),
          pltpu.VMEM((CH,), jnp.int32),
          pltpu.VMEM((CH, WID), jnp.float32),
          pltpu.VMEM((CH, WID), jnp.float32),
          pltpu.VMEM((CH, WID), jnp.float32),
          pltpu.VMEM((HID,), jnp.float32),
          pltpu.VMEM((SROWS, WID), jnp.float32),
          pltpu.VMEM_SHARED((NPAD, WID), jnp.float32),
          pltpu.SemaphoreType.DMA,
          pltpu.SemaphoreType.DMA,
      ],
  )
  def _sc_edge(xl_hbm, xr_hbm, src_hbm, dst_hbm, att_hbm, out_hbm,
               idx_s, idx_d, xlv, xrv, contrib, attv, stagev, acc,
               sem1, sem2):
    cid = lax.axis_index("c")
    sid = lax.axis_index("s")
    wkr = cid * NS + sid

    pltpu.sync_copy(att_hbm, attv)

    zvec = jnp.zeros((L,), jnp.float32)

    def _zero_row(i, carry):
        for c8 in range(WID // L):
            stagev[i, pl.ds(c8 * L, L)] = zvec
        return carry
    lax.fori_loop(0, SROWS, _zero_row, 0)

    for r in range(WB // SROWS):
        off = pl.multiple_of(sid * WB + r * SROWS, 8)
        pltpu.sync_copy(stagev, acc.at[pl.ds(off, SROWS)])

    plsc.subcore_barrier()

    atts = tuple(attv[pl.ds(16 * k, L)] for k in range(HID // L))
    dn = lax.GatherDimensionNumbers(
        offset_dims=(), collapsed_slice_dims=(0,), start_index_map=(0,))
    xor_idx = tuple(
        jnp.bitwise_xor(lax.iota(jnp.int32, L), d) for d in (1, 2, 4))
    ebase = wkr * EPW

    def _chunk(i, carry):
        base = pl.multiple_of(ebase + i * CH, 8)
        pltpu.sync_copy(src_hbm.at[pl.ds(base, CH)], idx_s)
        pltpu.sync_copy(dst_hbm.at[pl.ds(base, CH)], idx_d)
        cp1 = pltpu.async_copy(xl_hbm.at[idx_s], xlv, sem1)
        cp2 = pltpu.async_copy(xr_hbm.at[idx_d], xrv, sem2)
        cp1.wait()
        cp2.wait()

        def _edge(e, ecarry):
            for k in range(HID // L):
                xl = xlv[e, pl.ds(16 * k, L)]
                xr = xrv[e, pl.ds(16 * k, L)]
                s = xl + xr
                t = jnp.maximum(s, 0.2 * s) * atts[k]
                for xi in xor_idx:
                    t = t + lax.gather(
                        t, xi[:, None], dn, (1,),
                        mode=lax.GatherScatterMode.PROMISE_IN_BOUNDS)
                ex = jnp.exp(t)
                contrib[e, pl.ds(16 * k, L)] = ex * xl
                contrib[e, pl.ds(HID + 16 * k, L)] = ex
            return ecarry
        lax.fori_loop(0, CH, _edge, 0)

        pltpu.sync_copy(contrib, acc.at[idx_d], add=True)
        return carry
    lax.fori_loop(0, NCHUNK, _chunk, 0)

    plsc.subcore_barrier()

    for r in range(WB // SROWS):
        off = pl.multiple_of(sid * WB + r * SROWS, 8)
        pltpu.sync_copy(acc.at[pl.ds(off, SROWS)], stagev)
        pltpu.sync_copy(stagev, out_hbm.at[cid, pl.ds(off, SROWS)])

  return _sc_edge


def _elu(t):
    return jnp.where(t > 0, t, jnp.exp(jnp.minimum(t, 0.0)) - 1.0)


def _bn_elu(t, g, b):
    m = jnp.mean(t, axis=0, keepdims=True)
    v = jnp.mean((t - m) * (t - m), axis=0, keepdims=True)
    return _elu((t - m) * lax.rsqrt(v + 1e-5) * g + b)


def _combine_gat(p0, p1, h_prev, g, b):
    psum = (p0 + p1)[:N]
    num = psum[:, :HID]
    r = lax.broadcasted_iota(jnp.int32, (WID, HID), 0)
    cc = lax.broadcasted_iota(jnp.int32, (WID, HID), 1)
    sel = (r == HID + cc).astype(jnp.float32)
    den = jnp.dot(psum, sel, preferred_element_type=jnp.float32)
    gat = num / (den + 1e-16)
    return _bn_elu(gat + h_prev, g, b)


def _lr128(h, wl, wr):
    z = jnp.zeros((h.shape[0], WID - HID), jnp.float32)
    xl = jnp.dot(h, wl, preferred_element_type=jnp.float32)
    xr = jnp.dot(h, wr, preferred_element_type=jnp.float32)
    return (jnp.concatenate([xl, z], axis=1),
            jnp.concatenate([xr, z], axis=1))


def _pre_body(x_ref, wp_ref, g_ref, b_ref, wl_ref, wr_ref,
              h_ref, xl_ref, xr_ref):
    t = jnp.dot(x_ref[...], wp_ref[...], preferred_element_type=jnp.float32)
    h = _bn_elu(t, g_ref[...], b_ref[...])
    h_ref[...] = h
    xl_ref[...], xr_ref[...] = _lr128(h, wl_ref[...], wr_ref[...])


_tc_pre = pl.pallas_call(
    _pre_body,
    out_shape=[jax.ShapeDtypeStruct((N, HID), jnp.float32),
               jax.ShapeDtypeStruct((N, WID), jnp.float32),
               jax.ShapeDtypeStruct((N, WID), jnp.float32)],
)


def _mid_body(p0_ref, p1_ref, hp_ref, g_ref, b_ref, wl_ref, wr_ref,
              h_ref, xl_ref, xr_ref):
    h = _combine_gat(p0_ref[...], p1_ref[...], hp_ref[...],
                     g_ref[...], b_ref[...])
    h_ref[...] = h
    xl_ref[...], xr_ref[...] = _lr128(h, wl_ref[...], wr_ref[...])


_tc_mid = pl.pallas_call(
    _mid_body,
    out_shape=[jax.ShapeDtypeStruct((N, HID), jnp.float32),
               jax.ShapeDtypeStruct((N, WID), jnp.float32),
               jax.ShapeDtypeStruct((N, WID), jnp.float32)],
)


def _fin_body(p0_ref, p1_ref, hp_ref, g_ref, b_ref,
              w1_ref, b1_ref, w2_ref, b2_ref, w3_ref, b3_ref, w4_ref, b4_ref,
              o_ref):
    h = _combine_gat(p0_ref[...], p1_ref[...], hp_ref[...],
                     g_ref[...], b_ref[...])
    h = _elu(jnp.dot(h, w1_ref[...], preferred_element_type=jnp.float32)
             + b1_ref[...])
    h = _elu(jnp.dot(h, w2_ref[...], preferred_element_type=jnp.float32)
             + b2_ref[...])
    h = _elu(jnp.dot(h, w3_ref[...], preferred_element_type=jnp.float32)
             + b3_ref[...])
    o_ref[...] = (jnp.dot(h, w4_ref[...], preferred_element_type=jnp.float32)
                  + b4_ref[...])


_tc_fin = pl.pallas_call(
    _fin_body,
    out_shape=jax.ShapeDtypeStruct((N, 2), jnp.float32),
)


def kernel(x, edge_index, params):
    p = params
    src = edge_index[0]
    dst = edge_index[1]
    h, xl, xr = _tc_pre(x, p['W_pre'], p['bn0_g'], p['bn0_b'],
                        p['Wl1'], p['Wr1'])
    for l in (1, 2, 3):
        att_s = p['att%d' % l].reshape(H * C)
        parts = _make_sc_edge()(xl, xr, src, dst, att_s)
        h, xl, xr = _tc_mid(parts[0], parts[1], h,
                            p['bn%d_g' % l], p['bn%d_b' % l],
                            p['Wl%d' % (l + 1)], p['Wr%d' % (l + 1)])
    att_s = p['att4'].reshape(H * C)
    parts = _make_sc_edge()(xl, xr, src, dst, att_s)
    return _tc_fin(parts[0], parts[1], h, p['bn4_g'], p['bn4_b'],
                   p['W_p1'], p['b_p1'], p['W_p2'], p['b_p2'],
                   p['W_p3'], p['b_p3'], p['W_p4'], p['b_p4'])


# speedup vs baseline: 166.3238x; 1.9355x over previous
import functools

import jax
import jax.numpy as jnp
from jax import lax
from jax.experimental import pallas as pl
from jax.experimental.pallas import tpu as pltpu
from jax.experimental.pallas import tpu_sc as plsc

N = 10000
E = 320000
D_IN = 128
H = 8
C = 8
HID = 64
NC = 2
NS = 16
L = 16
NW = NC * NS
EPW = E // NW
CH = 40
TW = 128
NCHUNK = EPW // CH
WID = 128
NPAD = 10240
WB = NPAD // NS
SROWS = 64


CH2 = 2 * CH


@functools.cache
def _make_sc_edge():
  mesh = plsc.VectorSubcoreMesh(
      core_axis_name="c", subcore_axis_name="s", num_cores=NC, num_subcores=NS)

  @functools.partial(
      pl.kernel,
      out_type=jax.ShapeDtypeStruct((NC, NPAD, WID), jnp.float32),
      mesh=mesh,
      scratch_types=[
          [pltpu.VMEM((CH2,), jnp.int32) for _ in range(4)],
          [pltpu.VMEM((CH,), jnp.int32) for _ in range(4)],
          [pltpu.VMEM((CH2, TW), jnp.float32) for _ in range(2)],
          [pltpu.VMEM((CH, WID), jnp.float32) for _ in range(2)],
          pltpu.VMEM((HID,), jnp.float32),
          pltpu.VMEM_SHARED((NPAD, WID), jnp.float32),
          [pltpu.SemaphoreType.DMA for _ in range(4)],
          [pltpu.SemaphoreType.DMA for _ in range(2)],
          [pltpu.SemaphoreType.DMA for _ in range(2)],
      ],
  )
  def _sc_edge(t_hbm, src_hbm, dstn_hbm, dst_hbm, att_hbm, out_hbm,
               sidx, didx, xv, ctb, attv, acc, sem_i, sem_g, sem_sc):
    cid = lax.axis_index("c")
    sid = lax.axis_index("s")
    wkr = cid * NS + sid

    pltpu.sync_copy(att_hbm, attv)

    zvec = jnp.zeros((L,), jnp.float32)

    def _zero_row(i, carry):
        for c8 in range(WID // L):
            ctb[0][i, pl.ds(c8 * L, L)] = zvec
        return carry
    lax.fori_loop(0, CH, _zero_row, 0)

    for r in range(WB // CH):
        off = pl.multiple_of(sid * WB + r * CH, 8)
        pltpu.sync_copy(ctb[0], acc.at[pl.ds(off, CH)])

    plsc.subcore_barrier()

    atts = tuple(attv[pl.ds(16 * k, L)] for k in range(HID // L))
    dn = lax.GatherDimensionNumbers(
        offset_dims=(), collapsed_slice_dims=(0,), start_index_map=(0,))
    xor_idx = tuple(
        jnp.bitwise_xor(lax.iota(jnp.int32, L), d) for d in (1, 2, 4))
    ebase = wkr * EPW

    def _idx_descs(i, q):
        base = pl.multiple_of(ebase + i * CH, 8)
        return (pltpu.make_async_copy(src_hbm.at[pl.ds(base, CH)],
                                      sidx[q].at[pl.ds(0, CH)], sem_i[q]),
                pltpu.make_async_copy(dstn_hbm.at[pl.ds(base, CH)],
                                      sidx[q].at[pl.ds(CH, CH)], sem_i[q]),
                pltpu.make_async_copy(dst_hbm.at[pl.ds(base, CH)],
                                      didx[q], sem_i[q]))

    def _gather_desc(q, p):
        return pltpu.make_async_copy(t_hbm.at[sidx[q]], xv[p], sem_g[p])

    def _scatter_desc(p, q):
        return pltpu.make_async_copy(ctb[p], acc.at[didx[q]], sem_sc[p])

    for q in range(2):
        for d in _idx_descs(q, q):
            d.start()
    for p in range(2):
        for d in _idx_descs(p, p):
            d.wait()
        _gather_desc(p, p).start()

    NSUP = NCHUNK // 4
    REM = NCHUNK - NSUP * 4

    def _compute(p):
        def _edge(e2, ecarry):
            for u2 in range(2):
                e = e2 * 2 + u2
                for k in range(HID // L):
                    xl = xv[p][e, pl.ds(16 * k, L)]
                    xr = xv[p][CH + e, pl.ds(16 * k, L)]
                    s = xl + xr
                    t = jnp.maximum(s, 0.2 * s) * atts[k]
                    for xi in xor_idx:
                        t = t + lax.gather(
                            t, xi[:, None], dn, (1,),
                            mode=lax.GatherScatterMode.PROMISE_IN_BOUNDS)
                    ex = jnp.exp(t)
                    ctb[p][e, pl.ds(16 * k, L)] = ex * xl
                    ctb[p][e, pl.ds(HID + 16 * k, L)] = ex
            return ecarry
        lax.fori_loop(0, CH // 2, _edge, 0)

    def _slot(i, u, j):
        p = u % 2
        q = u % 4
        static = isinstance(i, int)
        _gather_desc(q, p).wait()

        def _wait_prev():
            _scatter_desc(p, (u + 2) % 4).wait()
        if u >= 2 or static:
            _wait_prev()
        else:
            pl.when(j > 0)(_wait_prev)

        def _refill():
            for d in _idx_descs(i + 2, (u + 2) % 4):
                d.start()
        if static:
            if i + 2 < NCHUNK:
                _refill()
        else:
            pl.when(i + 2 < NCHUNK)(_refill)

        _compute(p)
        pltpu.async_copy(ctb[p], acc.at[didx[q]], sem_sc[p], add=True)

        def _next_gather():
            for d in _idx_descs(i + 2, (u + 2) % 4):
                d.wait()
            _gather_desc((u + 2) % 4, p).start()
        if static:
            if i + 2 < NCHUNK:
                _next_gather()
        else:
            pl.when(i + 2 < NCHUNK)(_next_gather)

    def _quad(j, carry):
        for u in range(4):
            _slot(4 * j + u, u, j)
        return carry
    lax.fori_loop(0, NSUP, _quad, 0)
    for u in range(REM):
        _slot(NSUP * 4 + u, u, NSUP)

    _scatter_desc(0, (NCHUNK - 2) % 4).wait()
    _scatter_desc(1, (NCHUNK - 1) % 4).wait()

    plsc.subcore_barrier()

    descs = {}
    for r in range(WB // CH):
        b = r % 2
        if r >= 2:
            descs[r - 2].wait()
        off = pl.multiple_of(sid * WB + r * CH, 8)
        pltpu.sync_copy(acc.at[pl.ds(off, CH)], ctb[b])
        descs[r] = pltpu.async_copy(
            ctb[b], out_hbm.at[cid, pl.ds(off, CH)], sem_sc[b])
    descs[WB // CH - 2].wait()
    descs[WB // CH - 1].wait()

  return _sc_edge


def _elu(t):
    return jnp.where(t > 0, t, jnp.exp(jnp.minimum(t, 0.0)) - 1.0)


def _bn_elu(t, g, b):
    m = jnp.mean(t, axis=0, keepdims=True)
    v = jnp.mean((t - m) * (t - m), axis=0, keepdims=True)
    return _elu((t - m) * lax.rsqrt(v + 1e-5) * g + b)


def _combine_gat(pp, h_prev, g, b):
    psum = (pp[0] + pp[1])[:N]
    num = psum[:, :HID]
    r = lax.broadcasted_iota(jnp.int32, (WID, HID), 0)
    cc = lax.broadcasted_iota(jnp.int32, (WID, HID), 1)
    sel = (r == HID + cc).astype(jnp.float32)
    den = jnp.dot(psum, sel, preferred_element_type=jnp.float32)
    gat = num / (den + 1e-16)
    return _bn_elu(gat + h_prev, g, b)


def _lr128(h, wl, wr):
    z = jnp.zeros((h.shape[0], TW - HID), jnp.float32)
    xl = jnp.dot(h, wl, preferred_element_type=jnp.float32)
    xr = jnp.dot(h, wr, preferred_element_type=jnp.float32)
    return jnp.concatenate([
        jnp.concatenate([xl, z], axis=1),
        jnp.concatenate([xr, z], axis=1)], axis=0)


def _pre_body(x_ref, wp_ref, g_ref, b_ref, wl_ref, wr_ref,
              h_ref, t_ref):
    t = jnp.dot(x_ref[...], wp_ref[...], preferred_element_type=jnp.float32)
    h = _bn_elu(t, g_ref[...], b_ref[...])
    h_ref[...] = h
    t_ref[...] = _lr128(h, wl_ref[...], wr_ref[...])


_tc_pre = pl.pallas_call(
    _pre_body,
    out_shape=[jax.ShapeDtypeStruct((N, HID), jnp.float32),
               jax.ShapeDtypeStruct((2 * N, TW), jnp.float32)],
)


def _mid_body(pp_ref, hp_ref, g_ref, b_ref, wl_ref, wr_ref,
              h_ref, t_ref):
    h = _combine_gat(pp_ref[...], hp_ref[...], g_ref[...], b_ref[...])
    h_ref[...] = h
    t_ref[...] = _lr128(h, wl_ref[...], wr_ref[...])


_tc_mid = pl.pallas_call(
    _mid_body,
    out_shape=[jax.ShapeDtypeStruct((N, HID), jnp.float32),
               jax.ShapeDtypeStruct((2 * N, TW), jnp.float32)],
)


def _fin_body(pp_ref, hp_ref, g_ref, b_ref,
              w1_ref, b1_ref, w2_ref, b2_ref, w3_ref, b3_ref, w4_ref, b4_ref,
              o_ref):
    h = _combine_gat(pp_ref[...], hp_ref[...], g_ref[...], b_ref[...])
    h = _elu(jnp.dot(h, w1_ref[...], preferred_element_type=jnp.float32)
             + b1_ref[...])
    h = _elu(jnp.dot(h, w2_ref[...], preferred_element_type=jnp.float32)
             + b2_ref[...])
    h = _elu(jnp.dot(h, w3_ref[...], preferred_element_type=jnp.float32)
             + b3_ref[...])
    o_ref[...] = (jnp.dot(h, w4_ref[...], preferred_element_type=jnp.float32)
                  + b4_ref[...])


_tc_fin = pl.pallas_call(
    _fin_body,
    out_shape=jax.ShapeDtypeStruct((N, 2), jnp.float32),
)


def kernel(x, edge_index, params):
    p = params
    src = edge_index[0]
    dst = edge_index[1]
    dstn = dst + N
    h, tbl = _tc_pre(x, p['W_pre'], p['bn0_g'], p['bn0_b'],
                     p['Wl1'], p['Wr1'])
    for l in (1, 2, 3):
        att_s = p['att%d' % l].reshape(H * C)
        parts = _make_sc_edge()(tbl, src, dstn, dst, att_s)
        h, tbl = _tc_mid(parts, h,
                         p['bn%d_g' % l], p['bn%d_b' % l],
                         p['Wl%d' % (l + 1)], p['Wr%d' % (l + 1)])
    att_s = p['att4'].reshape(H * C)
    parts = _make_sc_edge()(tbl, src, dstn, dst, att_s)
    return _tc_fin(parts, h, p['bn4_g'], p['bn4_b'],
                   p['W_p1'], p['b_p1'], p['W_p2'], p['b_p2'],
                   p['W_p3'], p['b_p3'], p['W_p4'], p['b_p4'])
